# Initial kernel scaffold; baseline (speedup 1.0000x reference)
import jax
import jax.numpy as jnp
from jax.experimental import pallas as pl


def kernel(boxes, classification):
    raise NotImplementedError("write your pallas kernel here")


# speedup vs baseline: 7.0449x; 7.0449x over previous
import jax
import jax.numpy as jnp
from jax import lax
from jax.experimental import pallas as pl
from jax.experimental.pallas import tpu as pltpu

_NMS_THR = 0.5
_SCORE_THR = 0.05
_MAX_DET = 300
_NPAD = 5120
_CPAD = 24
_K = 512
_BIG = 2**30
_NEG_INF = float("-inf")


def _fd_kernel(s_in, bx_in, out_box, out_s, out_lab,
               s_w, rec_s, rec_x1, rec_y1, rec_x2, rec_y2,
               posv, scr_s, scr_lab, scr_box):
    C, N = _CPAD, _NPAD
    x1 = bx_in[0, 0:1, :]
    y1 = bx_in[0, 1:2, :]
    x2 = bx_in[0, 2:3, :]
    y2 = bx_in[0, 3:4, :]
    area_all = jnp.maximum(x2 - x1, 0.0) * jnp.maximum(y2 - y1, 0.0)

    s0 = s_in[0]
    s_w[:, :] = jnp.where(s0 > _SCORE_THR, s0, _NEG_INF)

    rec_s[:, :] = jnp.full((C, _K), _NEG_INF, jnp.float32)
    rec_x1[:, :] = jnp.zeros((C, _K), jnp.float32)
    rec_y1[:, :] = jnp.zeros((C, _K), jnp.float32)
    rec_x2[:, :] = jnp.zeros((C, _K), jnp.float32)
    rec_y2[:, :] = jnp.zeros((C, _K), jnp.float32)

    iota_n = lax.broadcasted_iota(jnp.int32, (C, N), 1)
    iota_k = lax.broadcasted_iota(jnp.int32, (C, _K), 1)

    def nms_step(t, carry):
        s = s_w[:, :]
        m = jnp.max(s, axis=1, keepdims=True)
        cand = jnp.where(s == m, iota_n, N)
        idx = jnp.min(cand, axis=1, keepdims=True)
        keep = m > _NEG_INF
        onehot = iota_n == idx
        sx1 = jnp.sum(jnp.where(onehot, x1, 0.0), axis=1, keepdims=True)
        sy1 = jnp.sum(jnp.where(onehot, y1, 0.0), axis=1, keepdims=True)
        sx2 = jnp.sum(jnp.where(onehot, x2, 0.0), axis=1, keepdims=True)
        sy2 = jnp.sum(jnp.where(onehot, y2, 0.0), axis=1, keepdims=True)
        xx1 = jnp.maximum(sx1, x1)
        yy1 = jnp.maximum(sy1, y1)
        xx2 = jnp.minimum(sx2, x2)
        yy2 = jnp.minimum(sy2, y2)
        inter = jnp.maximum(xx2 - xx1, 0.0) * jnp.maximum(yy2 - yy1, 0.0)
        area_sel = jnp.maximum(sx2 - sx1, 0.0) * jnp.maximum(sy2 - sy1, 0.0)
        union = area_sel + area_all - inter
        iou = jnp.where(union > 0.0, inter / union, 0.0)
        suppress = (iou > _NMS_THR) | onehot
        s_w[:, :] = jnp.where(suppress, _NEG_INF, s)
        slot = iota_k == t
        rec_s[:, :] = jnp.where(slot, jnp.where(keep, m, _NEG_INF), rec_s[:, :])
        rec_x1[:, :] = jnp.where(slot, jnp.where(keep, sx1, 0.0), rec_x1[:, :])
        rec_y1[:, :] = jnp.where(slot, jnp.where(keep, sy1, 0.0), rec_y1[:, :])
        rec_x2[:, :] = jnp.where(slot, jnp.where(keep, sx2, 0.0), rec_x2[:, :])
        rec_y2[:, :] = jnp.where(slot, jnp.where(keep, sy2, 0.0), rec_y2[:, :])
        return carry

    lax.fori_loop(0, _MAX_DET, nms_step, 0)

    posv[:, :] = (lax.broadcasted_iota(jnp.int32, (C, _K), 0) * _K
                  + lax.broadcasted_iota(jnp.int32, (C, _K), 1))

    def merge_step(t, carry):
        pv = posv[:, :]
        act = pv < _BIG
        vals = jnp.where(act, rec_s[:, :], _NEG_INF)
        m2 = jnp.max(vals)
        pc = jnp.where(vals == m2, pv, _BIG)
        p = jnp.min(pc)
        onehot = pv == p
        valid = m2 > _NEG_INF
        lab = jnp.where(valid, (p // _K).astype(jnp.int32), jnp.int32(-1))
        bx1 = jnp.sum(jnp.where(onehot, rec_x1[:, :], 0.0))
        by1 = jnp.sum(jnp.where(onehot, rec_y1[:, :], 0.0))
        bx2 = jnp.sum(jnp.where(onehot, rec_x2[:, :], 0.0))
        by2 = jnp.sum(jnp.where(onehot, rec_y2[:, :], 0.0))
        slot = iota_k[0:1, :] == t
        scr_s[0:1, :] = jnp.where(slot, m2, scr_s[0:1, :])
        scr_lab[0:1, :] = jnp.where(slot, lab, scr_lab[0:1, :])
        scr_box[pl.ds(t, 1), 0:4] = jnp.concatenate(
            [bx1.reshape(1, 1), by1.reshape(1, 1),
             bx2.reshape(1, 1), by2.reshape(1, 1)], axis=1)
        posv[:, :] = jnp.where(onehot, _BIG, pv)
        return carry

    lax.fori_loop(0, _MAX_DET, merge_step, 0)

    out_s[0, 0, :] = scr_s[0, 0:_MAX_DET]
    out_lab[0, 0, :] = scr_lab[0, 0:_MAX_DET]
    out_box[0, :, :] = scr_box[0:_MAX_DET, 0:4]


def kernel(boxes, classification):
    B, N, _ = boxes.shape
    C = classification.shape[2]
    s_t = jnp.transpose(classification, (0, 2, 1))
    s_p = jnp.pad(s_t, ((0, 0), (0, _CPAD - C), (0, _NPAD - N)))
    b_t = jnp.transpose(boxes, (0, 2, 1))
    b_p = jnp.pad(b_t, ((0, 0), (0, 4), (0, _NPAD - N)))

    out_box, out_s, out_lab = pl.pallas_call(
        _fd_kernel,
        grid=(B,),
        in_specs=[
            pl.BlockSpec((1, _CPAD, _NPAD), lambda b: (b, 0, 0)),
            pl.BlockSpec((1, 8, _NPAD), lambda b: (b, 0, 0)),
        ],
        out_specs=[
            pl.BlockSpec((1, _MAX_DET, 4), lambda b: (b, 0, 0)),
            pl.BlockSpec((1, 1, _MAX_DET), lambda b: (b, 0, 0)),
            pl.BlockSpec((1, 1, _MAX_DET), lambda b: (b, 0, 0)),
        ],
        out_shape=[
            jax.ShapeDtypeStruct((B, _MAX_DET, 4), jnp.float32),
            jax.ShapeDtypeStruct((B, 1, _MAX_DET), jnp.float32),
            jax.ShapeDtypeStruct((B, 1, _MAX_DET), jnp.int32),
        ],
        scratch_shapes=[
            pltpu.VMEM((_CPAD, _NPAD), jnp.float32),
            pltpu.VMEM((_CPAD, _K), jnp.float32),
            pltpu.VMEM((_CPAD, _K), jnp.float32),
            pltpu.VMEM((_CPAD, _K), jnp.float32),
            pltpu.VMEM((_CPAD, _K), jnp.float32),
            pltpu.VMEM((_CPAD, _K), jnp.float32),
            pltpu.VMEM((_CPAD, _K), jnp.int32),
            pltpu.VMEM((8, _K), jnp.float32),
            pltpu.VMEM((8, _K), jnp.int32),
            pltpu.VMEM((_K, 8), jnp.float32),
        ],
    )(s_p, b_p)

    return out_box, out_s.reshape(B, _MAX_DET), out_lab.reshape(B, _MAX_DET)
